# baseline (speedup 1.0000x reference)
import jax
import jax.numpy as jnp
from jax import lax
from jax.experimental import pallas as pl
from jax.experimental.pallas import tpu as pltpu
from jax.experimental.pallas import tpu_sc as plsc

VOCAB = 1000
B = 4096
T = 20
NC = 2
NS = 16
NW = NC * NS
BPW = B // NW
NT = 8
MAIN = 896


def _gather_body(table_hbm, idx_hbm, out_hbm, tail_hbm, idx_v, bufa, bufb,
                 bufc, bufd, g0, g1, g2, g3, w0, w1, w2, w3):
    buf = (bufa, bufb, bufc, bufd)
    gsem = (g0, g1, g2, g3)
    wsem = (w0, w1, w2, w3)

    sid = lax.axis_index("s")
    wid = sid * NC + lax.axis_index("c")
    base = wid * BPW
    pltpu.sync_copy(idx_hbm.at[pl.ds(base, BPW)], idx_v)

    def gather(j, b):
        return pltpu.make_async_copy(
            table_hbm.at[idx_v.at[j]], buf[b], gsem[b]
        )

    def write_main(j, b):
        return pltpu.make_async_copy(
            buf[b].reshape(T, NT * 128).at[:, pl.ds(0, MAIN)],
            out_hbm.at[base + j, :, pl.ds(0, MAIN)],
            wsem[b],
        )

    def write_tail(j, b):
        return pltpu.make_async_copy(
            buf[b].reshape(T, NT * 128).at[:, pl.ds(MAIN, 128)],
            tail_hbm.at[base + j],
            wsem[b],
        )

    for b in range(3):
        gather(b, b).start()

    def group(g, carry):
        for b in range(4):
            j = 4 * g + b
            bn = (b + 3) % 4
            if b == 0:
                @pl.when(g >= 1)
                def _():
                    write_main(j - 1, bn).wait()
                    write_tail(j - 1, bn).wait()
                    gather(j + 3, bn).start()

                @pl.when(g == 0)
                def _():
                    gather(j + 3, bn).start()
            else:
                write_main(j - 1, bn).wait()
                write_tail(j - 1, bn).wait()

                @pl.when(j + 3 < BPW)
                def _():
                    gather(j + 3, bn).start()
            gather(j, b).wait()
            write_main(j, b).start()
            write_tail(j, b).start()
        return carry

    lax.fori_loop(0, BPW // 4, group, 0)
    write_main(BPW - 1, (BPW - 1) % 4).wait()
    write_tail(BPW - 1, (BPW - 1) % 4).wait()


@jax.jit
def _run(idx, table3):
    mesh = plsc.VectorSubcoreMesh(core_axis_name="c", subcore_axis_name="s")
    out, tail = pl.kernel(
        _gather_body,
        out_type=(
            jax.ShapeDtypeStruct((B, T, VOCAB), jnp.float32),
            jax.ShapeDtypeStruct((B, T, 128), jnp.float32),
        ),
        mesh=mesh,
        scratch_types=[
            pltpu.VMEM((BPW, T), jnp.int32),
            pltpu.VMEM((T, NT, 128), jnp.float32),
            pltpu.VMEM((T, NT, 128), jnp.float32),
            pltpu.VMEM((T, NT, 128), jnp.float32),
            pltpu.VMEM((T, NT, 128), jnp.float32),
            pltpu.SemaphoreType.DMA,
            pltpu.SemaphoreType.DMA,
            pltpu.SemaphoreType.DMA,
            pltpu.SemaphoreType.DMA,
            pltpu.SemaphoreType.DMA,
            pltpu.SemaphoreType.DMA,
            pltpu.SemaphoreType.DMA,
            pltpu.SemaphoreType.DMA,
        ],
    )(table3, idx)
    return lax.dynamic_update_slice(
        out, lax.slice(tail, (0, 0, 0), (B, T, VOCAB - MAIN)), (0, 0, MAIN)
    )


def kernel(idx, token_embedding_table):
    table3 = jnp.pad(
        token_embedding_table, ((0, 0), (0, NT * 128 - VOCAB))
    ).reshape(VOCAB, NT, 128)
    return _run(idx, table3)


# speedup vs baseline: 1.0003x; 1.0003x over previous
import jax
import jax.numpy as jnp
from jax import lax
from jax.experimental import pallas as pl
from jax.experimental.pallas import tpu as pltpu
from jax.experimental.pallas import tpu_sc as plsc

VOCAB = 1000
B = 4096
T = 20
NC = 2
NS = 16
NW = NC * NS
BPW = B // NW
NT = 8
MAIN = 896


def _gather_body(table_hbm, idx_hbm, out_hbm, tail_hbm, idx_v, bufa, bufb,
                 g0, g1, w0, w1):
    buf = (bufa, bufb)
    gsem = (g0, g1)
    wsem = (w0, w1)

    sid = lax.axis_index("s")
    wid = sid * NC + lax.axis_index("c")
    base = wid * BPW
    pltpu.sync_copy(idx_hbm.at[pl.ds(base, BPW)], idx_v)

    def gather(j, b):
        return pltpu.make_async_copy(
            table_hbm.at[idx_v.at[j]], buf[b], gsem[b]
        )

    def write_main(j, b):
        return pltpu.make_async_copy(
            buf[b].reshape(T, NT * 128).at[:, pl.ds(0, MAIN)],
            out_hbm.at[base + j, :, pl.ds(0, MAIN)],
            wsem[b],
        )

    def write_tail(j, b):
        return pltpu.make_async_copy(
            buf[b].reshape(T, NT * 128).at[:, pl.ds(MAIN, 128)],
            tail_hbm.at[base + j],
            wsem[b],
        )

    gather(0, 0).start()
    gather(1, 1).start()

    def group(g, carry):
        for b in range(2):
            j = 2 * g + b
            gather(j, b).wait()

            @pl.when(g >= 1)
            def _():
                write_main(j - 2, b).wait()
                write_tail(j - 2, b).wait()

            write_main(j, b).start()
            write_tail(j, b).start()

            @pl.when(g < BPW // 2 - 1)
            def _():
                gather(j + 2, b).start()
        return carry

    lax.fori_loop(0, BPW // 2, group, 0)
    for j, b in ((BPW - 2, 0), (BPW - 1, 1)):
        write_main(j, b).wait()
        write_tail(j, b).wait()


@jax.jit
def _run(idx, table3):
    mesh = plsc.VectorSubcoreMesh(core_axis_name="c", subcore_axis_name="s")
    out, tail = pl.kernel(
        _gather_body,
        out_type=(
            jax.ShapeDtypeStruct((B, T, VOCAB), jnp.float32),
            jax.ShapeDtypeStruct((B, T, 128), jnp.float32),
        ),
        mesh=mesh,
        scratch_types=[
            pltpu.VMEM((BPW, T), jnp.int32),
            pltpu.VMEM((T, NT, 128), jnp.float32),
            pltpu.VMEM((T, NT, 128), jnp.float32),
            pltpu.SemaphoreType.DMA,
            pltpu.SemaphoreType.DMA,
            pltpu.SemaphoreType.DMA,
            pltpu.SemaphoreType.DMA,
        ],
    )(table3, idx)
    return lax.dynamic_update_slice(
        out, lax.slice(tail, (0, 0, 0), (B, T, VOCAB - MAIN)), (0, 0, MAIN)
    )


def kernel(idx, token_embedding_table):
    table3 = jnp.pad(
        token_embedding_table, ((0, 0), (0, NT * 128 - VOCAB))
    ).reshape(VOCAB, NT, 128)
    return _run(idx, table3)
